# baseline (speedup 1.0000x reference)
import math

import jax
import jax.numpy as jnp
from jax import lax
from jax.experimental import pallas as pl
from jax.experimental.pallas import tpu as pltpu
from jax.experimental.pallas import tpu_sc as plsc

D = 64
NUM_WORKERS = 32
CHUNK = 128
SCALE = math.sqrt(64.0)
LANES = 16


def _emb_body(x_hbm, tab_hbm, out_hbm, idx_v, gidx_v, rows_v, out_v, sem):
    c = lax.axis_index("c")
    s = lax.axis_index("s")
    wid = s * 2 + c
    per_w = x_hbm.shape[1]
    nchunk = per_w // CHUNK
    pltpu.sync_copy(x_hbm.at[wid], idx_v)

    iota = lax.iota(jnp.int32, LANES)

    def chunk_body(j, carry):
        def gidx_body(k, carry2):
            sl = pl.ds(j * CHUNK + k * LANES, LANES)
            gidx_v[pl.ds(k * LANES, LANES)] = lax.shift_right_logical(
                idx_v[sl], 1
            )
            return carry2

        lax.fori_loop(0, CHUNK // LANES, gidx_body, 0, unroll=True)

        pltpu.async_copy(tab_hbm.at[gidx_v], rows_v, sem).wait()

        def group_body(g, carry2):
            par16 = idx_v[pl.ds(j * CHUNK + g * LANES, LANES)]
            col16 = lax.shift_left(
                lax.bitwise_and(par16, jnp.int32(1)), jnp.int32(6)
            )
            for i in range(LANES):
                r = g * LANES + i
                col0 = col16[i]
                for cc in range(D // LANES):
                    sl = pl.ds(col0 + cc * LANES, LANES)
                    out_v[r, pl.ds(cc * LANES, LANES)] = rows_v[r, sl] * SCALE
            return carry2

        lax.fori_loop(0, CHUNK // LANES, group_body, 0)

        base = wid * per_w + j * CHUNK
        pltpu.sync_copy(out_v, out_hbm.at[pl.ds(base, CHUNK)])
        return carry

    lax.fori_loop(0, nchunk, chunk_body, 0)


@jax.jit
def kernel(x, table):
    rows, cols = x.shape
    total = rows * cols
    per_w = total // NUM_WORKERS
    xw = x.reshape(NUM_WORKERS, per_w)
    tab2 = table.reshape(table.shape[0] // 2, 2 * table.shape[1])

    mesh = plsc.VectorSubcoreMesh(core_axis_name="c", subcore_axis_name="s")
    out = pl.kernel(
        _emb_body,
        out_type=jax.ShapeDtypeStruct((total, D), jnp.float32),
        mesh=mesh,
        scratch_types=[
            pltpu.VMEM((per_w,), jnp.int32),
            pltpu.VMEM((CHUNK,), jnp.int32),
            pltpu.VMEM((CHUNK, 2 * D), jnp.float32),
            pltpu.VMEM((CHUNK, D), jnp.float32),
            pltpu.SemaphoreType.DMA,
        ],
    )(xw, tab2)
    return out.reshape(rows, cols, D)


# speedup vs baseline: 1.1594x; 1.1594x over previous
import math

import jax
import jax.numpy as jnp
from jax import lax
from jax.experimental import pallas as pl
from jax.experimental.pallas import tpu as pltpu
from jax.experimental.pallas import tpu_sc as plsc

D = 64
NUM_WORKERS = 32
SEQ = 200
BPW = 4096 // NUM_WORKERS
SCALE = math.sqrt(64.0)
LANES = 16
SPLITS = ((0, 128), (128, 72))


def _emb_body(x_hbm, tab_hbm, out_hbm, idx_v, gidx_v, rows_v, out_v,
              gsem0, gsem1, osem0, osem1):
    c = lax.axis_index("c")
    s = lax.axis_index("s")
    wid = s * 2 + c
    gsems = (gsem0, gsem1)
    osems = (osem0, osem1)

    def stage_batch(j, slot):
        pltpu.sync_copy(
            x_hbm.at[wid * BPW + j], idx_v.at[slot]
        )
        for k in range(13):
            off = k * LANES if k < 12 else SEQ - LANES
            gidx_v[slot, pl.ds(off, LANES)] = lax.shift_right_logical(
                idx_v[slot, pl.ds(off, LANES)], 1
            )

    def fire_gather(slot):
        for off, n in SPLITS:
            pltpu.async_copy(
                tab_hbm.at[gidx_v.at[slot].at[pl.ds(off, n)]],
                rows_v.at[slot].at[pl.ds(off, n)],
                gsems[slot],
            )

    def wait_gather(slot):
        for off, n in SPLITS:
            pltpu.make_async_copy(
                tab_hbm.at[gidx_v.at[slot].at[pl.ds(off, n)]],
                rows_v.at[slot].at[pl.ds(off, n)],
                gsems[slot],
            ).wait()

    def fire_out(j, slot):
        pltpu.async_copy(
            out_v.at[slot], out_hbm.at[wid * BPW + j], osems[slot]
        )

    def wait_out(j, slot):
        pltpu.make_async_copy(
            out_v.at[slot], out_hbm.at[wid * BPW + j], osems[slot]
        ).wait()

    def select_scale(slot):
        def group_body(k, carry):
            par16 = idx_v[slot, pl.ds(k * LANES, LANES)]
            col16 = lax.shift_left(
                lax.bitwise_and(par16, jnp.int32(1)), jnp.int32(6)
            )
            for i in range(LANES):
                r = k * LANES + i
                col0 = col16[i]
                for cc in range(D // LANES):
                    out_v[slot, r, pl.ds(cc * LANES, LANES)] = (
                        rows_v[slot, r, pl.ds(col0 + cc * LANES, LANES)]
                        * SCALE
                    )
            return carry

        lax.fori_loop(0, 12, group_body, 0)
        par16 = idx_v[slot, pl.ds(SEQ - LANES, LANES)]
        col16 = lax.shift_left(
            lax.bitwise_and(par16, jnp.int32(1)), jnp.int32(6)
        )
        for i in range(8, LANES):
            r = SEQ - LANES + i
            col0 = col16[i]
            for cc in range(D // LANES):
                out_v[slot, r, pl.ds(cc * LANES, LANES)] = (
                    rows_v[slot, r, pl.ds(col0 + cc * LANES, LANES)]
                    * SCALE
                )

    stage_batch(0, 0)
    fire_gather(0)

    def body(j2, carry):
        for b in range(2):
            j = j2 * 2 + b
            nj = j + 1
            if b == 0:
                stage_batch(nj, 1)
                fire_gather(1)
            else:
                @pl.when(j2 < (BPW // 2) - 1)
                def _():
                    stage_batch(nj, 0)
                    fire_gather(0)
            wait_gather(b)

            @pl.when(j2 >= 1)
            def _():
                wait_out(j - 2, b)

            select_scale(b)
            fire_out(j, b)
        return carry

    lax.fori_loop(0, BPW // 2, body, 0)
    wait_out(BPW - 2, 0)
    wait_out(BPW - 1, 1)


@jax.jit
def kernel(x, table):
    rows, cols = x.shape
    mesh = plsc.VectorSubcoreMesh(core_axis_name="c", subcore_axis_name="s")
    tab2 = table.reshape(table.shape[0] // 2, 2 * table.shape[1])
    out = pl.kernel(
        _emb_body,
        out_type=jax.ShapeDtypeStruct((rows, cols, D), jnp.float32),
        mesh=mesh,
        scratch_types=[
            pltpu.VMEM((2, SEQ), jnp.int32),
            pltpu.VMEM((2, SEQ), jnp.int32),
            pltpu.VMEM((2, SEQ, 2 * D), jnp.float32),
            pltpu.VMEM((2, SEQ, D), jnp.float32),
            pltpu.SemaphoreType.DMA,
            pltpu.SemaphoreType.DMA,
            pltpu.SemaphoreType.DMA,
            pltpu.SemaphoreType.DMA,
        ],
    )(x, tab2)
    return out


# speedup vs baseline: 1.3607x; 1.1736x over previous
import math

import jax
import jax.numpy as jnp
from jax import lax
from jax.experimental import pallas as pl
from jax.experimental.pallas import tpu as pltpu
from jax.experimental.pallas import tpu_sc as plsc

D = 64
NUM_WORKERS = 32
SEQ = 200
BPW = 4096 // NUM_WORKERS
SCALE = math.sqrt(64.0)
LANES = 16
SPLITS = ((0, 128), (128, 72))


def _emb_body(x_hbm, tab_hbm, out_hbm, idx_v, rows_v,
              isem0, isem1, gsem0, gsem1, osem0, osem1):
    c = lax.axis_index("c")
    s = lax.axis_index("s")
    wid = s * 2 + c
    isems = (isem0, isem1)
    gsems = (gsem0, gsem1)
    osems = (osem0, osem1)

    def fire_stage(j, slot):
        pltpu.async_copy(x_hbm.at[wid * BPW + j], idx_v.at[slot],
                         isems[slot])

    def wait_stage(slot):
        pltpu.make_async_copy(x_hbm.at[wid * BPW], idx_v.at[slot],
                              isems[slot]).wait()

    def fire_gather(slot):
        for off, n in SPLITS:
            pltpu.async_copy(
                tab_hbm.at[idx_v.at[slot].at[pl.ds(off, n)]],
                rows_v.at[slot].at[pl.ds(off, n)],
                gsems[slot],
            )

    def wait_gather(slot):
        for off, n in SPLITS:
            pltpu.make_async_copy(
                tab_hbm.at[idx_v.at[slot].at[pl.ds(off, n)]],
                rows_v.at[slot].at[pl.ds(off, n)],
                gsems[slot],
            ).wait()

    def fire_out(j, slot):
        pltpu.async_copy(
            rows_v.at[slot], out_hbm.at[wid * BPW + j], osems[slot]
        )

    def wait_out(j, slot):
        pltpu.make_async_copy(
            rows_v.at[slot], out_hbm.at[wid * BPW + j], osems[slot]
        ).wait()

    def scale(slot):
        def group_body(k, carry):
            for i in range(8):
                r = k * 8 + i
                for cc in range(D // LANES):
                    sl = pl.ds(cc * LANES, LANES)
                    rows_v[slot, r, sl] = rows_v[slot, r, sl] * SCALE
            return carry

        lax.fori_loop(0, SEQ // 8, group_body, 0)

    fire_stage(0, 0)
    wait_stage(0)
    fire_gather(0)
    fire_stage(1, 1)

    def body(j2, carry):
        for b in range(2):
            j = j2 * 2 + b
            other = 1 - b

            @pl.when(j2 * 2 + b < BPW - 1)
            def _():
                wait_stage(other)

                if b == 0:
                    @pl.when(j2 >= 1)
                    def _():
                        wait_out(j - 1, other)
                else:
                    wait_out(j - 1, other)

                fire_gather(other)

                @pl.when(j2 * 2 + b < BPW - 2)
                def _():
                    fire_stage(j + 2, b)

            wait_gather(b)
            scale(b)
            fire_out(j, b)
        return carry

    lax.fori_loop(0, BPW // 2, body, 0)
    wait_out(BPW - 2, 0)
    wait_out(BPW - 1, 1)


@jax.jit
def kernel(x, table):
    rows, cols = x.shape
    mesh = plsc.VectorSubcoreMesh(core_axis_name="c", subcore_axis_name="s")
    out = pl.kernel(
        _emb_body,
        out_type=jax.ShapeDtypeStruct((rows, cols, D), jnp.float32),
        mesh=mesh,
        compiler_params=pltpu.CompilerParams(use_tc_tiling_on_sc=False),
        scratch_types=[
            pltpu.VMEM((2, SEQ), jnp.int32),
            pltpu.VMEM((2, SEQ, D), jnp.float32),
            pltpu.SemaphoreType.DMA,
            pltpu.SemaphoreType.DMA,
            pltpu.SemaphoreType.DMA,
            pltpu.SemaphoreType.DMA,
            pltpu.SemaphoreType.DMA,
            pltpu.SemaphoreType.DMA,
        ],
    )(x, table)
    return out
